# Initial kernel scaffold; baseline (speedup 1.0000x reference)
import jax
import jax.numpy as jnp
from jax.experimental import pallas as pl


def kernel(x, edge_index, W1, b1, W2, b2):
    raise NotImplementedError("write your pallas kernel here")


# speedup vs baseline: 52.2939x; 52.2939x over previous
import jax
import jax.numpy as jnp
from jax import lax
from jax.experimental import pallas as pl
from jax.experimental.pallas import tpu as pltpu
from jax.experimental.pallas import tpu_sc as plsc

N0 = 100000
NPAD = 100352
RPT_N = NPAD // 16
E0 = 3200000
SUP = 8
NSUP = 98
RPT_E = SUP * NSUP
EROWS = 32 * RPT_E
EPAD = EROWS * 128

_MESH = plsc.VectorSubcoreMesh(core_axis_name="c", subcore_axis_name="s",
                               num_cores=2, num_subcores=16)


def _deg_body(dst2, degp, idxd, ones_v, zbuf, accd, semd):
    cid = lax.axis_index("c")
    sid = lax.axis_index("s")
    wid = sid * 2 + cid
    zv = jnp.zeros((16,), jnp.float32)
    ov = jnp.ones((16,), jnp.float32)

    def fill_z(k, _):
        zbuf[pl.ds(k * 16, 16)] = zv
        return 0
    lax.fori_loop(0, RPT_N // 16, fill_z, 0)

    def fill_o(i, carry):
        def fill_o2(j, c2):
            ones_v[i, pl.ds(j * 16, 16)] = ov
            return c2
        return lax.fori_loop(0, 128 // 16, fill_o2, carry)
    lax.fori_loop(0, SUP, fill_o, 0)

    nb = sid * RPT_N
    pltpu.sync_copy(zbuf, accd.at[pl.ds(nb, RPT_N)])
    plsc.subcore_barrier()

    eb = wid * RPT_E

    def step(s, carry):
        pltpu.sync_copy(dst2.at[pl.ds(eb + s * SUP, SUP)], idxd)
        cps = [pltpu.async_copy(ones_v.at[b], accd.at[idxd.at[b]], semd,
                                add=True)
               for b in range(SUP)]
        for cp in cps:
            cp.wait()
        return carry
    lax.fori_loop(0, NSUP, step, 0)
    plsc.subcore_barrier()
    pltpu.sync_copy(accd.at[pl.ds(nb, RPT_N)], degp.at[cid, pl.ds(nb, RPT_N)])


_deg_call = pl.kernel(
    _deg_body,
    out_type=jax.ShapeDtypeStruct((2, NPAD), jnp.float32),
    mesh=_MESH,
    scratch_types=[
        pltpu.VMEM((SUP, 128), jnp.int32),
        pltpu.VMEM((SUP, 128), jnp.float32),
        pltpu.VMEM((RPT_N,), jnp.float32),
        pltpu.VMEM_SHARED((NPAD,), jnp.float32),
        pltpu.SemaphoreType.DMA,
    ],
    compiler_params=pltpu.CompilerParams(use_tc_tiling_on_sc=False),
)


def _agg_body(table, src2, dst2, aggp, idxs, idxd, rows, zbuf, acc, sem, sem2):
    cid = lax.axis_index("c")
    sid = lax.axis_index("s")
    wid = sid * 2 + cid
    zv = jnp.zeros((16,), jnp.float32)

    def fill_z(i, carry):
        zbuf[i, :] = zv
        return carry
    lax.fori_loop(0, 128, fill_z, 0)

    nb = sid * RPT_N

    def zcopy(j, carry):
        pltpu.sync_copy(zbuf, acc.at[pl.ds(nb + j * 128, 128)])
        return carry
    lax.fori_loop(0, RPT_N // 128, zcopy, 0)
    plsc.subcore_barrier()

    eb = wid * RPT_E

    def step(s, carry):
        rb = eb + s * SUP
        pltpu.sync_copy(src2.at[pl.ds(rb, SUP)], idxs)
        pltpu.sync_copy(dst2.at[pl.ds(rb, SUP)], idxd)
        gs = [pltpu.async_copy(table.at[idxs.at[b]], rows.at[b], sem)
              for b in range(SUP)]
        for cp in gs:
            cp.wait()
        ss = [pltpu.async_copy(rows.at[b], acc.at[idxd.at[b]], sem2, add=True)
              for b in range(SUP)]
        for cp in ss:
            cp.wait()
        return carry
    lax.fori_loop(0, NSUP, step, 0)
    plsc.subcore_barrier()
    pltpu.sync_copy(acc.at[pl.ds(nb, RPT_N)], aggp.at[cid, pl.ds(nb, RPT_N)])


_agg_call = pl.kernel(
    _agg_body,
    out_type=jax.ShapeDtypeStruct((2, NPAD, 16), jnp.float32),
    mesh=_MESH,
    scratch_types=[
        pltpu.VMEM((SUP, 128), jnp.int32),
        pltpu.VMEM((SUP, 128), jnp.int32),
        pltpu.VMEM((SUP, 128, 16), jnp.float32),
        pltpu.VMEM((128, 16), jnp.float32),
        pltpu.VMEM_SHARED((NPAD, 16), jnp.float32),
        pltpu.SemaphoreType.DMA,
        pltpu.SemaphoreType.DMA,
    ],
    compiler_params=pltpu.CompilerParams(use_tc_tiling_on_sc=False),
)


BR = 2048
GRID = NPAD // BR


def _prep_body(d0, d1, x, w1, y1, dinv):
    deg = 1.0 + d0[:] + d1[:]
    di = lax.rsqrt(deg)
    dinv[:] = di
    xl = jnp.dot(x[:], w1[:], preferred_element_type=jnp.float32)
    y1[:] = xl * di[:, None]


_prep_call = pl.pallas_call(
    _prep_body,
    grid=(GRID,),
    in_specs=[
        pl.BlockSpec((BR,), lambda i: (i,)),
        pl.BlockSpec((BR,), lambda i: (i,)),
        pl.BlockSpec((BR, 5), lambda i: (i, 0)),
        pl.BlockSpec((5, 16), lambda i: (0, 0)),
    ],
    out_specs=[
        pl.BlockSpec((BR, 16), lambda i: (i, 0)),
        pl.BlockSpec((BR,), lambda i: (i,)),
    ],
    out_shape=[
        jax.ShapeDtypeStruct((NPAD, 16), jnp.float32),
        jax.ShapeDtypeStruct((NPAD,), jnp.float32),
    ],
)


def _mid_body(a0, a1, y1, dinv, b1, z2):
    i = pl.program_id(0)
    di = dinv[:]
    h = di[:, None] * (a0[:] + a1[:] + y1[:]) + b1[:][None, :]
    h = jnp.maximum(h, 0.0)
    rows = i * BR + lax.broadcasted_iota(jnp.int32, (BR, 1), 0)
    z2[:] = jnp.where(rows < N0, di[:, None] * h, 0.0)


_mid_call = pl.pallas_call(
    _mid_body,
    grid=(GRID,),
    in_specs=[
        pl.BlockSpec((BR, 16), lambda i: (i, 0)),
        pl.BlockSpec((BR, 16), lambda i: (i, 0)),
        pl.BlockSpec((BR, 16), lambda i: (i, 0)),
        pl.BlockSpec((BR,), lambda i: (i,)),
        pl.BlockSpec((16,), lambda i: (0,)),
    ],
    out_specs=pl.BlockSpec((BR, 16), lambda i: (i, 0)),
    out_shape=jax.ShapeDtypeStruct((NPAD, 16), jnp.float32),
)


def _fin_body(a0, a1, z2, dinv, w2, b2, o):
    g = dinv[:][:, None] * (a0[:] + a1[:] + z2[:])
    t = jnp.dot(g, w2[:], preferred_element_type=jnp.float32) + b2[:][None, :]
    m = jnp.max(t, axis=1, keepdims=True)
    s = t - m
    lse = jnp.log(jnp.sum(jnp.exp(s), axis=1, keepdims=True))
    o[:] = s - lse


_fin_call = pl.pallas_call(
    _fin_body,
    grid=(GRID,),
    in_specs=[
        pl.BlockSpec((BR, 16), lambda i: (i, 0)),
        pl.BlockSpec((BR, 16), lambda i: (i, 0)),
        pl.BlockSpec((BR, 16), lambda i: (i, 0)),
        pl.BlockSpec((BR,), lambda i: (i,)),
        pl.BlockSpec((16, 2), lambda i: (0, 0)),
        pl.BlockSpec((2,), lambda i: (0,)),
    ],
    out_specs=pl.BlockSpec((BR, 2), lambda i: (i, 0)),
    out_shape=jax.ShapeDtypeStruct((NPAD, 2), jnp.float32),
)


def kernel(x, edge_index, W1, b1, W2, b2):
    pad_e = EPAD - E0
    pad_idx = jnp.full((pad_e,), N0, jnp.int32)
    src2 = jnp.concatenate([edge_index[0], pad_idx]).reshape(EROWS, 128)
    dst2 = jnp.concatenate([edge_index[1], pad_idx]).reshape(EROWS, 128)
    x_pad = jnp.zeros((NPAD, 5), jnp.float32).at[:N0].set(x)

    degp = _deg_call(dst2)
    y1, dinv = _prep_call(degp[0], degp[1], x_pad, W1)
    a1 = _agg_call(y1, src2, dst2)
    z2 = _mid_call(a1[0], a1[1], y1, dinv, b1)
    a2 = _agg_call(z2, src2, dst2)
    out = _fin_call(a2[0], a2[1], z2, dinv, W2, b2)
    return out[:N0]
